# baseline (speedup 1.0000x reference)
import functools

import jax
import jax.numpy as jnp
import numpy as np
from jax import lax
from jax.experimental import pallas as pl
from jax.experimental.pallas import tpu as pltpu
from jax.experimental.pallas import tpu_sc as plsc

_IDXW = 128
_GPC = 4
_CHUNK = _IDXW * _GPC
_DP = 56
_D = 50
_GROUP = 8


def _compaction_map():
    cmap = np.zeros((2 * _D // 2, 16), np.int32)
    for j in range(25):
        p = j * 16 + np.arange(16)
        cmap[2 * j] = p // _D
        cmap[2 * j + 1] = p % _D
    return jnp.asarray(cmap)


def _make_gather(n_flat: int):
    info = plsc.get_sparse_core_info()
    nw = info.num_cores * info.num_subcores
    assert n_flat % (nw * _CHUNK) == 0
    per_w = n_flat // nw
    n_rows_w = per_w // _IDXW
    n_outer = per_w // _CHUNK
    n_inner = _CHUNK // _GROUP

    mesh = plsc.VectorSubcoreMesh(core_axis_name="c", subcore_axis_name="s")

    @functools.partial(
        pl.kernel,
        out_type=jax.ShapeDtypeStruct((n_flat * _D,), jnp.float32),
        mesh=mesh,
        compiler_params=pltpu.CompilerParams(
            use_tc_tiling_on_sc=False, needs_layout_passes=False),
        scratch_types=[
            pltpu.VMEM((n_rows_w, _IDXW), jnp.int32),
            pltpu.VMEM((2 * _D // 2, 16), jnp.int32),
            pltpu.VMEM((_CHUNK, _DP), jnp.float32),
            pltpu.VMEM((_CHUNK * _D,), jnp.float32),
        ] + [pltpu.SemaphoreType.DMA] * _GPC,
    )
    def gather_kernel(idx_hbm, cmap_hbm, table_hbm, out_hbm,
                      idx_v, cmap_v, rows_v, comp_v, *sems):
        wid = lax.axis_index("s") * info.num_cores + lax.axis_index("c")
        pltpu.sync_copy(cmap_hbm, cmap_v)
        pltpu.sync_copy(idx_hbm.at[pl.ds(wid * n_rows_w, n_rows_w)], idx_v)

        def body(c, _):
            copies = []
            for g in range(_GPC):
                cp = pltpu.make_async_copy(
                    table_hbm.at[idx_v.at[c * _GPC + g]],
                    rows_v.at[pl.ds(g * _IDXW, _IDXW)],
                    sems[g],
                )
                cp.start()
                copies.append(cp)
            for cp in copies:
                cp.wait()

            def inner(g, rbase):
                for j in range(25):
                    rv = cmap_v[2 * j] + rbase
                    cv = cmap_v[2 * j + 1]
                    vals = plsc.load_gather(rows_v, [rv, cv])
                    comp_v[pl.ds(g * (_GROUP * _D) + j * 16, 16)] = vals
                return rbase + _GROUP

            lax.fori_loop(0, n_inner, inner, jnp.zeros((16,), jnp.int32),
                          unroll=False)

            base = (wid * per_w + c * _CHUNK) * _D
            pltpu.sync_copy(comp_v, out_hbm.at[pl.ds(base, _CHUNK * _D)])
            return ()

        lax.fori_loop(0, n_outer, body, (), unroll=False)

    return gather_kernel


def kernel(input_ids, table):
    b, l = input_ids.shape
    vocab, dim = table.shape
    n_flat = b * l
    tpad = jnp.pad(table, ((0, 0), (0, _DP - dim)))
    idx = input_ids.reshape(n_flat // _IDXW, _IDXW)
    out_flat = _make_gather(n_flat)(idx, _compaction_map(), tpad)
    return out_flat.reshape(b, l, dim)


# speedup vs baseline: 1.5509x; 1.5509x over previous
import functools

import jax
import jax.numpy as jnp
from jax import lax
from jax.experimental import pallas as pl
from jax.experimental.pallas import tpu as pltpu
from jax.experimental.pallas import tpu_sc as plsc

_IDXW = 128
_GPC = 8
_CHUNK = _IDXW * _GPC
_DP = 56


def _make_gather(n_flat: int):
    info = plsc.get_sparse_core_info()
    nw = info.num_cores * info.num_subcores
    assert n_flat % (nw * _CHUNK) == 0
    per_w = n_flat // nw
    n_rows_w = per_w // _IDXW
    n_outer = per_w // _CHUNK

    mesh = plsc.VectorSubcoreMesh(core_axis_name="c", subcore_axis_name="s")

    @functools.partial(
        pl.kernel,
        out_type=jax.ShapeDtypeStruct((n_flat, _DP), jnp.float32),
        mesh=mesh,
        compiler_params=pltpu.CompilerParams(use_tc_tiling_on_sc=False),
        scratch_types=[
            pltpu.VMEM((n_rows_w, _IDXW), jnp.int32),
            pltpu.VMEM((_CHUNK, _DP), jnp.float32),
        ] + [pltpu.SemaphoreType.DMA] * _GPC,
    )
    def gather_kernel(idx_hbm, table_hbm, out_hbm, idx_v, rows_v, *sems):
        wid = lax.axis_index("s") * info.num_cores + lax.axis_index("c")
        pltpu.sync_copy(idx_hbm.at[pl.ds(wid * n_rows_w, n_rows_w)], idx_v)

        def body(c, _):
            copies = []
            for g in range(_GPC):
                cp = pltpu.make_async_copy(
                    table_hbm.at[idx_v.at[c * _GPC + g]],
                    rows_v.at[pl.ds(g * _IDXW, _IDXW)],
                    sems[g],
                )
                cp.start()
                copies.append(cp)
            for cp in copies:
                cp.wait()
            base = wid * per_w + c * _CHUNK
            pltpu.sync_copy(rows_v, out_hbm.at[pl.ds(base, _CHUNK)])
            return ()

        lax.fori_loop(0, n_outer, body, (), unroll=False)

    return gather_kernel


def kernel(input_ids, table):
    b, l = input_ids.shape
    vocab, dim = table.shape
    n_flat = b * l
    tpad = jnp.pad(table, ((0, 0), (0, _DP - dim)))
    idx = input_ids.reshape(n_flat // _IDXW, _IDXW)
    outp = _make_gather(n_flat)(idx, tpad)
    return outp[:, :dim].reshape(b, l, dim)
